# baseline (speedup 1.0000x reference)
import functools

import jax
import jax.numpy as jnp
import numpy as np
from jax.experimental import pallas as pl
from jax.experimental.pallas import tpu as pltpu

_INF = float(np.float32(3.0e38))


def _roll_l(x, s):
  n = x.shape[-1]
  s = s % n
  if s == 0:
    return x
  return jnp.concatenate([x[..., n - s:], x[..., :n - s]], axis=-1)


def _sparsify_body(w_ref, wsoft_ref, stats_ref, colabs, keepinf, keep01):
  p = pl.program_id(0)
  t = pl.program_id(1)
  n = w_ref.shape[-1]
  ch = min(n, 512)

  @pl.when((p == 0) & (t == 0))
  def _init():
    colabs[...] = jnp.zeros_like(colabs)
    stats_ref[0] = 0.0
    stats_ref[1] = 0.0

  @pl.when(p == 0)
  def _acc_colabs():
    colabs[...] += jnp.sum(jnp.abs(w_ref[...]), axis=0, keepdims=True)

  @pl.when((p == 1) & (t == 0))
  def _make_keep():
    c = colabs[...]
    ob = jax.lax.broadcasted_iota(jnp.int32, (1, n), 1) % 8
    rank = jnp.zeros((1, n), jnp.float32)
    for k in range(1, 8):
      p_nw = _roll_l(c, -k)
      p_w = _roll_l(c, 8 - k)
      nw = ob < (8 - k)
      beat_nw = jnp.where(p_nw > c, 1.0, 0.0)
      beat_w = jnp.where(p_w >= c, 1.0, 0.0)
      rank += jnp.where(nw, beat_nw, beat_w)
    keepinf[...] = jnp.where(rank < 3.5, 0.0, _INF)
    keep01[...] = jnp.where(rank < 3.5, 1.0, 0.0)

  @pl.when(p == 1)
  def _soft():
    obc = jax.lax.broadcasted_iota(jnp.int32, (1, ch), 1) % 8
    up_guard = [jnp.where(obc < (8 - d), 0.0, _INF) for d in (1, 2, 4)]
    dn_guard = [jnp.where((obc & d) != 0, 0.0, _INF) for d in (1, 2, 4)]
    num_p = jnp.float32(0.0)
    den_p = jnp.float32(0.0)
    for c0 in range(0, n, ch):
      w = w_ref[:, c0:c0 + ch]
      ki = keepinf[:, c0:c0 + ch]
      m1 = jnp.abs(w) + ki
      b1 = jnp.maximum(_roll_l(m1, -1), up_guard[0])
      m2 = jnp.maximum(m1, b1)
      m1 = jnp.minimum(m1, b1)
      for i, d in enumerate((2, 4)):
        b1 = jnp.maximum(_roll_l(m1, -d), up_guard[i + 1])
        b2 = jnp.maximum(_roll_l(m2, -d), up_guard[i + 1])
        hi = jnp.maximum(m1, b1)
        m1 = jnp.minimum(m1, b1)
        m2 = jnp.minimum(hi, jnp.minimum(m2, b2))
      thr = m2
      for i, d in enumerate((1, 2, 4)):
        thr = jnp.minimum(thr, _roll_l(thr, d) + dn_guard[i])
      ws = (w - jnp.clip(w, -thr, thr)) * keep01[:, c0:c0 + ch]
      num_p += jnp.sum(w * ws)
      den_p += jnp.sum(ws * ws)
      wsoft_ref[:, c0:c0 + ch] = ws.astype(jnp.bfloat16)
    stats_ref[0] += num_p
    stats_ref[1] += den_p


def _cast_body(x_ref, o_ref):
  o_ref[...] = x_ref[...].astype(jnp.bfloat16)


def _matmul_body(beta_ref, x_ref, w_ref, y_ref):
  acc = jax.lax.dot_general(
      x_ref[...], w_ref[...],
      dimension_numbers=(((1,), (1,)), ((), ())),
      preferred_element_type=jnp.float32)
  y_ref[...] = acc * beta_ref[0]


def _pick(n, t):
  return t if n % t == 0 else n


def _impl(x, W, interpret):
  b, s, d = x.shape
  out_f, in_f = W.shape
  xm = x.reshape(b * s, d)
  mm = b * s

  to = _pick(out_f, 128)
  nt = out_f // to

  wsoft, stats = pl.pallas_call(
      _sparsify_body,
      grid=(2, nt),
      in_specs=[pl.BlockSpec((to, in_f), lambda p, t: (t, 0))],
      out_specs=[
          pl.BlockSpec((to, in_f), lambda p, t: (t, 0)),
          pl.BlockSpec(memory_space=pltpu.SMEM),
      ],
      out_shape=[
          jax.ShapeDtypeStruct((out_f, in_f), jnp.bfloat16),
          jax.ShapeDtypeStruct((2,), jnp.float32),
      ],
      scratch_shapes=[
          pltpu.VMEM((1, in_f), jnp.float32),
          pltpu.VMEM((1, in_f), jnp.float32),
          pltpu.VMEM((1, in_f), jnp.float32),
      ],
      compiler_params=pltpu.CompilerParams(
          dimension_semantics=("arbitrary", "arbitrary"),
          vmem_limit_bytes=100 * 1024 * 1024,
      ),
      interpret=interpret,
  )(W)

  num = stats[0]
  den = stats[1]
  beta = jnp.where(den > 0, num / den, jnp.float32(1.0))
  beta = jax.lax.stop_gradient(beta).reshape(1)

  bc = _pick(mm, 512)
  xb = pl.pallas_call(
      _cast_body,
      grid=(mm // bc,),
      in_specs=[pl.BlockSpec((bc, in_f), lambda i: (i, 0))],
      out_specs=pl.BlockSpec((bc, in_f), lambda i: (i, 0)),
      out_shape=jax.ShapeDtypeStruct((mm, in_f), jnp.bfloat16),
      compiler_params=pltpu.CompilerParams(
          dimension_semantics=("parallel",),
          vmem_limit_bytes=50 * 1024 * 1024,
      ),
      interpret=interpret,
  )(xm)

  bm = _pick(mm, 1024)
  bn = _pick(out_f, 1024)
  y = pl.pallas_call(
      _matmul_body,
      grid=(mm // bm, out_f // bn),
      in_specs=[
          pl.BlockSpec(memory_space=pltpu.SMEM),
          pl.BlockSpec((bm, in_f), lambda i, j: (i, 0)),
          pl.BlockSpec((bn, in_f), lambda i, j: (j, 0)),
      ],
      out_specs=pl.BlockSpec((bm, bn), lambda i, j: (i, j)),
      out_shape=jax.ShapeDtypeStruct((mm, out_f), jnp.float32),
      compiler_params=pltpu.CompilerParams(
          dimension_semantics=("parallel", "arbitrary"),
          vmem_limit_bytes=60 * 1024 * 1024,
      ),
      interpret=interpret,
  )(beta, xb, wsoft)

  return y.reshape(b, s, out_f)


def kernel(x, W):
  return _impl(x, W, False)


# speedup vs baseline: 1.0950x; 1.0950x over previous
import functools

import jax
import jax.numpy as jnp
import numpy as np
from jax.experimental import pallas as pl
from jax.experimental.pallas import tpu as pltpu

_INF = float(np.float32(3.0e38))


def _roll_l(x, s):
  n = x.shape[-1]
  s = s % n
  if s == 0:
    return x
  return jnp.concatenate([x[..., n - s:], x[..., :n - s]], axis=-1)


def _sparsify_body(w_ref, wsoft_ref, stats_ref, colabs, keepinf, keep01):
  p = pl.program_id(0)
  t = pl.program_id(1)
  n = w_ref.shape[-1]
  ch = min(n, 512)

  @pl.when((p == 0) & (t == 0))
  def _init():
    colabs[...] = jnp.zeros_like(colabs)
    stats_ref[0] = 0.0
    stats_ref[1] = 0.0

  @pl.when(p == 0)
  def _acc_colabs():
    colabs[...] += jnp.sum(jnp.abs(w_ref[...]), axis=0, keepdims=True)

  @pl.when((p == 1) & (t == 0))
  def _make_keep():
    c = colabs[...]
    ob = jax.lax.broadcasted_iota(jnp.int32, (1, n), 1) % 8
    rank = jnp.zeros((1, n), jnp.float32)
    for k in range(1, 8):
      p_nw = _roll_l(c, -k)
      p_w = _roll_l(c, 8 - k)
      nw = ob < (8 - k)
      beat_nw = jnp.where(p_nw > c, 1.0, 0.0)
      beat_w = jnp.where(p_w >= c, 1.0, 0.0)
      rank += jnp.where(nw, beat_nw, beat_w)
    keepinf[...] = jnp.where(rank < 3.5, 0.0, _INF)
    keep01[...] = jnp.where(rank < 3.5, 1.0, 0.0)

  @pl.when(p == 1)
  def _soft():
    obc = jax.lax.broadcasted_iota(jnp.int32, (1, n), 1) % 8
    up_guard = [jnp.where(obc < (8 - d), 0.0, _INF) for d in (1, 2, 4)]
    dn_guard = [jnp.where((obc & d) != 0, 0.0, _INF) for d in (1, 2, 4)]
    w = w_ref[...]
    m1 = jnp.abs(w) + keepinf[...]
    b1 = jnp.maximum(_roll_l(m1, -1), up_guard[0])
    m2 = jnp.maximum(m1, b1)
    m1 = jnp.minimum(m1, b1)
    for i, d in enumerate((2, 4)):
      b1 = jnp.maximum(_roll_l(m1, -d), up_guard[i + 1])
      b2 = jnp.maximum(_roll_l(m2, -d), up_guard[i + 1])
      hi = jnp.maximum(m1, b1)
      m1 = jnp.minimum(m1, b1)
      m2 = jnp.minimum(hi, jnp.minimum(m2, b2))
    thr = m2
    for i, d in enumerate((1, 2, 4)):
      thr = jnp.minimum(thr, _roll_l(thr, d) + dn_guard[i])
    ws = (w - jnp.clip(w, -thr, thr)) * keep01[...]
    stats_ref[0] += jnp.sum(w * ws)
    stats_ref[1] += jnp.sum(ws * ws)
    wsoft_ref[...] = ws.astype(jnp.bfloat16)


def _cast_body(x_ref, o_ref):
  o_ref[...] = x_ref[...].astype(jnp.bfloat16)


def _matmul_body(beta_ref, x_ref, w_ref, y_ref):
  acc = jax.lax.dot_general(
      x_ref[...], w_ref[...],
      dimension_numbers=(((1,), (1,)), ((), ())),
      preferred_element_type=jnp.float32)
  y_ref[...] = acc * beta_ref[0]


def _pick(n, t):
  return t if n % t == 0 else n


def _impl(x, W, interpret):
  b, s, d = x.shape
  out_f, in_f = W.shape
  xm = x.reshape(b * s, d)
  mm = b * s

  to = _pick(out_f, 128)
  nt = out_f // to

  wsoft, stats = pl.pallas_call(
      _sparsify_body,
      grid=(2, nt),
      in_specs=[pl.BlockSpec((to, in_f), lambda p, t: (t, 0))],
      out_specs=[
          pl.BlockSpec((to, in_f), lambda p, t: (t, 0)),
          pl.BlockSpec(memory_space=pltpu.SMEM),
      ],
      out_shape=[
          jax.ShapeDtypeStruct((out_f, in_f), jnp.bfloat16),
          jax.ShapeDtypeStruct((2,), jnp.float32),
      ],
      scratch_shapes=[
          pltpu.VMEM((1, in_f), jnp.float32),
          pltpu.VMEM((1, in_f), jnp.float32),
          pltpu.VMEM((1, in_f), jnp.float32),
      ],
      compiler_params=pltpu.CompilerParams(
          dimension_semantics=("arbitrary", "arbitrary"),
          vmem_limit_bytes=100 * 1024 * 1024,
      ),
      interpret=interpret,
  )(W)

  num = stats[0]
  den = stats[1]
  beta = jnp.where(den > 0, num / den, jnp.float32(1.0))
  beta = jax.lax.stop_gradient(beta).reshape(1)

  bc = _pick(mm, 512)
  xb = pl.pallas_call(
      _cast_body,
      grid=(mm // bc,),
      in_specs=[pl.BlockSpec((bc, in_f), lambda i: (i, 0))],
      out_specs=pl.BlockSpec((bc, in_f), lambda i: (i, 0)),
      out_shape=jax.ShapeDtypeStruct((mm, in_f), jnp.bfloat16),
      compiler_params=pltpu.CompilerParams(
          dimension_semantics=("parallel",),
          vmem_limit_bytes=50 * 1024 * 1024,
      ),
      interpret=interpret,
  )(xm)

  bm = _pick(mm, 1024)
  bn = _pick(out_f, 1024)
  y = pl.pallas_call(
      _matmul_body,
      grid=(mm // bm, out_f // bn),
      in_specs=[
          pl.BlockSpec(memory_space=pltpu.SMEM),
          pl.BlockSpec((bm, in_f), lambda i, j: (i, 0)),
          pl.BlockSpec((bn, in_f), lambda i, j: (j, 0)),
      ],
      out_specs=pl.BlockSpec((bm, bn), lambda i, j: (i, j)),
      out_shape=jax.ShapeDtypeStruct((mm, out_f), jnp.float32),
      compiler_params=pltpu.CompilerParams(
          dimension_semantics=("parallel", "arbitrary"),
          vmem_limit_bytes=60 * 1024 * 1024,
      ),
      interpret=interpret,
  )(beta, xb, wsoft)

  return y.reshape(b, s, out_f)


def kernel(x, W):
  return _impl(x, W, False)


# speedup vs baseline: 1.3790x; 1.2593x over previous
import functools

import jax
import jax.numpy as jnp
import numpy as np
from jax.experimental import pallas as pl
from jax.experimental.pallas import tpu as pltpu

_INF = float(np.float32(3.0e38))


def _roll_l(x, s):
  n = x.shape[-1]
  s = s % n
  if s == 0:
    return x
  return jnp.concatenate([x[..., n - s:], x[..., :n - s]], axis=-1)


def _sparsify_body(w_ref, wsoft_ref, stats_ref, colmap_ref, colabs, keepinf,
                   keep01):
  p = pl.program_id(0)
  t = pl.program_id(1)
  n = w_ref.shape[-1]
  ch = min(n, 512)

  @pl.when((p == 0) & (t == 0))
  def _init():
    colabs[...] = jnp.zeros_like(colabs)
    stats_ref[0] = 0.0
    stats_ref[1] = 0.0

  @pl.when(p == 0)
  def _acc_colabs():
    colabs[...] += jnp.sum(jnp.abs(w_ref[...]), axis=0, keepdims=True)

  @pl.when((p == 1) & (t == 0))
  def _make_keep():
    c = colabs[...]
    ob = jax.lax.broadcasted_iota(jnp.int32, (1, n), 1) % 8
    rank = jnp.zeros((1, n), jnp.float32)
    for k in range(1, 8):
      p_nw = _roll_l(c, -k)
      p_w = _roll_l(c, 8 - k)
      nw = ob < (8 - k)
      beat_nw = jnp.where(p_nw > c, 1.0, 0.0)
      beat_w = jnp.where(p_w >= c, 1.0, 0.0)
      rank += jnp.where(nw, beat_nw, beat_w)
    keepinf[...] = jnp.where(rank < 3.5, 0.0, _INF)
    k01 = jnp.where(rank < 3.5, 1.0, 0.0)
    keep01[...] = k01
    li = jax.lax.broadcasted_iota(jnp.int32, (1, n), 1)
    pre = jnp.zeros((1, n), jnp.float32)
    for k in range(1, 8):
      g = jnp.where(ob >= k, 1.0, 0.0)
      pre += g * _roll_l(k01, k)
    b4 = ((li >> 3) * 4).astype(jnp.float32)
    colmap_ref[...] = jnp.where(rank < 3.5, b4 + pre, -1.0e9)

  @pl.when(p == 1)
  def _soft():
    obc = jax.lax.broadcasted_iota(jnp.int32, (1, n), 1) % 8
    up_guard = [jnp.where(obc < (8 - d), 0.0, _INF) for d in (1, 2, 4)]
    dn_guard = [jnp.where((obc & d) != 0, 0.0, _INF) for d in (1, 2, 4)]
    w = w_ref[...]
    m1 = jnp.abs(w) + keepinf[...]
    b1 = jnp.maximum(_roll_l(m1, -1), up_guard[0])
    m2 = jnp.maximum(m1, b1)
    m1 = jnp.minimum(m1, b1)
    for i, d in enumerate((2, 4)):
      b1 = jnp.maximum(_roll_l(m1, -d), up_guard[i + 1])
      b2 = jnp.maximum(_roll_l(m2, -d), up_guard[i + 1])
      hi = jnp.maximum(m1, b1)
      m1 = jnp.minimum(m1, b1)
      m2 = jnp.minimum(hi, jnp.minimum(m2, b2))
    thr = m2
    for i, d in enumerate((1, 2, 4)):
      thr = jnp.minimum(thr, _roll_l(thr, d) + dn_guard[i])
    ws = (w - jnp.clip(w, -thr, thr)) * keep01[...]
    stats_ref[0] += jnp.sum(w * ws)
    stats_ref[1] += jnp.sum(ws * ws)
    wsoft_ref[...] = ws.astype(jnp.bfloat16)


def _cast_body(x_ref, o_ref):
  o_ref[...] = x_ref[...].astype(jnp.bfloat16)


def _pbuild_body(cm_ref, p_ref):
  br, npc = p_ref.shape
  base = pl.program_id(0) * (br // 2)
  cm = cm_ref[...]
  for j in range(npc // 128):
    tgt = cm - (base + 128 * j).astype(jnp.float32)
    col = jax.lax.broadcasted_iota(jnp.int32, (br, 128), 1).astype(jnp.float32)
    p_ref[:, 128 * j:128 * (j + 1)] = jnp.where(col == tgt, 1.0,
                                                0.0).astype(jnp.bfloat16)


def _compress_body(x_ref, p_ref, o_ref):
  k = x_ref.shape[1]
  npc = p_ref.shape[1]
  ch = 2 * npc
  xb = x_ref[...].astype(jnp.bfloat16)
  for c in range(k // ch):
    acc = jax.lax.dot_general(
        xb[:, ch * c:ch * (c + 1)], p_ref[ch * c:ch * (c + 1), :],
        dimension_numbers=(((1,), (0,)), ((), ())),
        preferred_element_type=jnp.float32)
    o_ref[:, npc * c:npc * (c + 1)] = acc.astype(jnp.bfloat16)


def _matmul_body(beta_ref, x_ref, w_ref, y_ref):
  acc = jax.lax.dot_general(
      x_ref[...], w_ref[...],
      dimension_numbers=(((1,), (1,)), ((), ())),
      preferred_element_type=jnp.float32)
  y_ref[...] = acc * beta_ref[0]


def _pick(n, t):
  return t if n % t == 0 else n


def _impl(x, W, interpret):
  b, s, d = x.shape
  out_f, in_f = W.shape
  xm = x.reshape(b * s, d)
  mm = b * s

  to = _pick(out_f, 128)
  nt = out_f // to

  wsoft, stats, colmap = pl.pallas_call(
      _sparsify_body,
      grid=(2, nt),
      in_specs=[pl.BlockSpec((to, in_f), lambda p, t: (t, 0))],
      out_specs=[
          pl.BlockSpec((to, in_f), lambda p, t: (t, 0)),
          pl.BlockSpec(memory_space=pltpu.SMEM),
          pl.BlockSpec((1, in_f), lambda p, t: (0, 0)),
      ],
      out_shape=[
          jax.ShapeDtypeStruct((out_f, in_f), jnp.bfloat16),
          jax.ShapeDtypeStruct((2,), jnp.float32),
          jax.ShapeDtypeStruct((1, in_f), jnp.float32),
      ],
      scratch_shapes=[
          pltpu.VMEM((1, in_f), jnp.float32),
          pltpu.VMEM((1, in_f), jnp.float32),
          pltpu.VMEM((1, in_f), jnp.float32),
      ],
      compiler_params=pltpu.CompilerParams(
          dimension_semantics=("arbitrary", "arbitrary"),
          vmem_limit_bytes=100 * 1024 * 1024,
      ),
      interpret=interpret,
  )(W)

  num = stats[0]
  den = stats[1]
  beta = jnp.where(den > 0, num / den, jnp.float32(1.0))
  beta = jax.lax.stop_gradient(beta).reshape(1)

  kc = in_f // 2
  ch_in = _pick(in_f, 512)
  npc = ch_in // 2
  cm_rep = jnp.broadcast_to(colmap.reshape(in_f, 1), (in_f, 128))
  pmat = pl.pallas_call(
      _pbuild_body,
      grid=(in_f // ch_in,),
      in_specs=[pl.BlockSpec((ch_in, 128), lambda c: (c, 0))],
      out_specs=pl.BlockSpec((ch_in, npc), lambda c: (c, 0)),
      out_shape=jax.ShapeDtypeStruct((in_f, npc), jnp.bfloat16),
      compiler_params=pltpu.CompilerParams(
          dimension_semantics=("arbitrary",),
          vmem_limit_bytes=50 * 1024 * 1024,
      ),
      interpret=interpret,
  )(cm_rep)

  def _compress(src, rows):
    br = _pick(rows, 512)
    return pl.pallas_call(
        _compress_body,
        grid=(rows // br,),
        in_specs=[
            pl.BlockSpec((br, in_f), lambda i: (i, 0)),
            pl.BlockSpec((in_f, npc), lambda i: (0, 0)),
        ],
        out_specs=pl.BlockSpec((br, kc), lambda i: (i, 0)),
        out_shape=jax.ShapeDtypeStruct((rows, kc), jnp.bfloat16),
        compiler_params=pltpu.CompilerParams(
            dimension_semantics=("parallel",),
            vmem_limit_bytes=60 * 1024 * 1024,
        ),
        interpret=interpret,
    )(src, pmat)

  xc = _compress(xm, mm)
  wc = _compress(wsoft, out_f)

  bm = _pick(mm, 1024)
  bn = _pick(out_f, 1024)
  y = pl.pallas_call(
      _matmul_body,
      grid=(mm // bm, out_f // bn),
      in_specs=[
          pl.BlockSpec(memory_space=pltpu.SMEM),
          pl.BlockSpec((bm, kc), lambda i, j: (i, 0)),
          pl.BlockSpec((bn, kc), lambda i, j: (j, 0)),
      ],
      out_specs=pl.BlockSpec((bm, bn), lambda i, j: (i, j)),
      out_shape=jax.ShapeDtypeStruct((mm, out_f), jnp.float32),
      compiler_params=pltpu.CompilerParams(
          dimension_semantics=("parallel", "arbitrary"),
          vmem_limit_bytes=60 * 1024 * 1024,
      ),
      interpret=interpret,
  )(beta, xc, wc)

  return y.reshape(b, s, out_f)


def kernel(x, W):
  return _impl(x, W, False)
